# baseline (speedup 1.0000x reference)
import functools

import jax
import jax.numpy as jnp
from jax import lax
from jax.experimental import pallas as pl
from jax.experimental.pallas import tpu as pltpu
from jax.experimental.pallas import tpu_sc as plsc

N_NODES = 10000
N_EDGES = 320000
D_IN = 128
D_HID = 64
D_OUT = 2

NC = 2
NS = 16
NW = NC * NS
CHUNK = 128
EDGES_PER_TILE = N_EDGES // NW
FULL_CHUNKS = EDGES_PER_TILE // CHUNK
TAIL = EDGES_PER_TILE - FULL_CHUNKS * CHUNK
N_PAD = 10112
ROWS_PER_TILE = N_PAD // NS
CW = 16
ROW_BLK = 5000


def _sc_scatter(with_cnt):
    mesh = plsc.VectorSubcoreMesh(core_axis_name="c", subcore_axis_name="s")
    agg_t = jax.ShapeDtypeStruct((NC, N_PAD, D_HID), jnp.float32)
    out_type = [agg_t] if with_cnt else agg_t
    scratch = (
        [pltpu.VMEM((EDGES_PER_TILE,), jnp.int32),
         pltpu.VMEM((EDGES_PER_TILE,), jnp.int32)]
        + [pltpu.VMEM((CHUNK, D_HID), jnp.float32)] * 4
        + [pltpu.VMEM((TAIL, D_HID), jnp.float32),
           pltpu.VMEM_SHARED((N_PAD, D_HID), jnp.float32)]
        + [pltpu.SemaphoreType.DMA] * 8
    )
    if with_cnt:
        out_type.append(jax.ShapeDtypeStruct((NC, N_PAD, CW), jnp.float32))
        scratch += [
            pltpu.VMEM((CHUNK, CW), jnp.float32),
            pltpu.VMEM_SHARED((N_PAD, CW), jnp.float32),
            pltpu.SemaphoreType.DMA,
            pltpu.SemaphoreType.DMA,
        ]

    def body(*refs):
        if with_cnt:
            (p, ei, ones_h, z64, z16, agg_o, cnt_o,
             sv, dv, r0_, r1_, r2_, r3_, rT, acc,
             g0, g1, g2, g3, s0, s1, s2, s3,
             ones_v, cacc, semCA, semCB) = refs
        else:
            (p, ei, z64, agg_o,
             sv, dv, r0_, r1_, r2_, r3_, rT, acc,
             g0, g1, g2, g3, s0, s1, s2, s3) = refs
        rv = [r0_, r1_, r2_, r3_]
        gs = [g0, g1, g2, g3]
        ss = [s0, s1, s2, s3]
        cs = None if not with_cnt else [semCA, semCB]

        cid = lax.axis_index("c")
        sid = lax.axis_index("s")
        wid = cid * NS + sid
        r0 = sid * ROWS_PER_TILE
        base = wid * EDGES_PER_TILE

        pltpu.sync_copy(z64, acc.at[pl.ds(r0, ROWS_PER_TILE)])
        if with_cnt:
            pltpu.sync_copy(z16, cacc.at[pl.ds(r0, ROWS_PER_TILE)])
            pltpu.sync_copy(ones_h, ones_v)
        pltpu.sync_copy(ei.at[0].at[pl.ds(base, EDGES_PER_TILE)], sv)
        pltpu.sync_copy(ei.at[1].at[pl.ds(base, EDGES_PER_TILE)], dv)
        plsc.subcore_barrier()

        def g_start(j, b):
            pltpu.async_copy(p.at[sv.at[pl.ds(j * CHUNK, CHUNK)]],
                             rv[b], gs[b])

        def g_drain(j, b):
            pltpu.make_async_copy(p.at[sv.at[pl.ds(j * CHUNK, CHUNK)]],
                                  rv[b], gs[b]).wait()

        def r_fire(j, b):
            pltpu.async_copy(rv[b], acc.at[dv.at[pl.ds(j * CHUNK, CHUNK)]],
                             ss[b], add=True)

        def r_wait(j, b):
            pltpu.make_async_copy(rv[b],
                                  acc.at[dv.at[pl.ds(j * CHUNK, CHUNK)]],
                                  ss[b]).wait()

        def c_fire(j, m):
            pltpu.async_copy(ones_v, cacc.at[dv.at[pl.ds(j * CHUNK, CHUNK)]],
                             cs[m], add=True)

        def c_wait(j, m):
            pltpu.make_async_copy(
                ones_v, cacc.at[dv.at[pl.ds(j * CHUNK, CHUNK)]], cs[m]).wait()

        tail_ds = pl.ds(FULL_CHUNKS * CHUNK, TAIL)

        g_start(0, 0)
        g_start(1, 1)
        if with_cnt:
            c_fire(0, 0)
            c_fire(1, 1)
        g_drain(0, 0)
        r_fire(0, 0)
        g_start(2, 2)
        g_drain(1, 1)
        r_fire(1, 1)
        g_start(3, 3)

        def step(j, b, m):
            g_drain(j, b)
            r_fire(j, b)
            if with_cnt:
                c_wait(j - 2, m)
                c_fire(j, m)
            r_wait(j - 2, (b + 2) % 4)
            g_start(j + 2, (b + 2) % 4)

        @pl.loop(0, (FULL_CHUNKS - 6) // 4)
        def _(it):
            j0 = 2 + it * 4
            for k in range(4):
                step(j0 + k, (2 + k) % 4, k % 2)

        jE = FULL_CHUNKS - 4
        g_drain(jE, 2)
        r_fire(jE, 2)
        if with_cnt:
            c_wait(jE - 2, 0)
            c_fire(jE, 0)
        r_wait(jE - 2, 0)
        g_start(jE + 2, 0)

        g_drain(jE + 1, 3)
        r_fire(jE + 1, 3)
        if with_cnt:
            c_wait(jE - 1, 1)
            c_fire(jE + 1, 1)
        r_wait(jE - 1, 1)
        g_start(jE + 3, 1)

        g_drain(jE + 2, 0)
        r_fire(jE + 2, 0)
        if with_cnt:
            c_wait(jE, 0)
            c_fire(jE + 2, 0)
        r_wait(jE, 2)
        pltpu.async_copy(p.at[sv.at[tail_ds]], rT, gs[2])

        g_drain(jE + 3, 1)
        r_fire(jE + 3, 1)
        if with_cnt:
            c_wait(jE + 1, 1)
            c_fire(jE + 3, 1)
        r_wait(jE + 1, 3)

        pltpu.make_async_copy(p.at[sv.at[tail_ds]], rT, gs[2]).wait()
        pltpu.sync_copy(rT, acc.at[dv.at[tail_ds]], add=True)
        if with_cnt:
            c_wait(jE + 2, 0)
            c_wait(jE + 3, 1)
            pltpu.sync_copy(ones_v.at[pl.ds(0, TAIL)],
                            cacc.at[dv.at[tail_ds]], add=True)
        r_wait(jE + 2, 0)
        r_wait(jE + 3, 1)

        plsc.subcore_barrier()
        pltpu.sync_copy(acc.at[pl.ds(r0, ROWS_PER_TILE)],
                        agg_o.at[cid].at[pl.ds(r0, ROWS_PER_TILE)])
        if with_cnt:
            pltpu.sync_copy(cacc.at[pl.ds(r0, ROWS_PER_TILE)],
                            cnt_o.at[cid].at[pl.ds(r0, ROWS_PER_TILE)])

    cp = pltpu.CompilerParams(use_tc_tiling_on_sc=False)
    return pl.kernel(body, out_type=out_type, mesh=mesh, scratch_types=scratch,
                     compiler_params=cp)


def _dense2(x, Wl, Wr, b2d):
    def tc_body(x_ref, wl_ref, wr_ref, b_ref, p_ref, q_ref):
        xb = x_ref[...]
        p_ref[...] = jnp.dot(xb, wl_ref[...],
                             preferred_element_type=jnp.float32,
                             precision=lax.Precision.HIGHEST)
        q_ref[...] = jnp.dot(xb, wr_ref[...],
                             preferred_element_type=jnp.float32,
                             precision=lax.Precision.HIGHEST) + b_ref[...]

    return pl.pallas_call(
        tc_body,
        grid=(N_NODES // ROW_BLK,),
        in_specs=[pl.BlockSpec((ROW_BLK, D_IN), lambda i: (i, 0)),
                  pl.BlockSpec((D_IN, D_HID), lambda i: (0, 0)),
                  pl.BlockSpec((D_IN, D_HID), lambda i: (0, 0)),
                  pl.BlockSpec((1, D_HID), lambda i: (0, 0))],
        out_specs=[pl.BlockSpec((ROW_BLK, D_HID), lambda i: (i, 0)),
                   pl.BlockSpec((ROW_BLK, D_HID), lambda i: (i, 0))],
        out_shape=[jax.ShapeDtypeStruct((N_NODES, D_HID), jnp.float32)] * 2,
    )(x, Wl, Wr, b2d)


def _mid(aggp, cntp, Q1, W2l, W2r, b2d):
    def tc_body(a_ref, c_ref, q_ref, wl_ref, wr_ref, b_ref, p_ref, q2_ref):
        a = a_ref[0] + a_ref[1]
        cnt = c_ref[0, :, 0:1] + c_ref[1, :, 0:1]
        inv = 1.0 / jnp.maximum(cnt, 1.0)
        h = jnp.maximum(a * inv + q_ref[...], 0.0)
        p_ref[...] = jnp.dot(h, wl_ref[...],
                             preferred_element_type=jnp.float32,
                             precision=lax.Precision.HIGHEST)
        q2_ref[...] = jnp.dot(h, wr_ref[...],
                              preferred_element_type=jnp.float32,
                              precision=lax.Precision.HIGHEST) + b_ref[...]

    return pl.pallas_call(
        tc_body,
        grid=(N_NODES // ROW_BLK,),
        in_specs=[pl.BlockSpec((NC, ROW_BLK, D_HID), lambda i: (0, i, 0)),
                  pl.BlockSpec((NC, ROW_BLK, CW), lambda i: (0, i, 0)),
                  pl.BlockSpec((ROW_BLK, D_HID), lambda i: (i, 0)),
                  pl.BlockSpec((D_HID, D_HID), lambda i: (0, 0)),
                  pl.BlockSpec((D_HID, D_HID), lambda i: (0, 0)),
                  pl.BlockSpec((1, D_HID), lambda i: (0, 0))],
        out_specs=[pl.BlockSpec((ROW_BLK, D_HID), lambda i: (i, 0)),
                   pl.BlockSpec((ROW_BLK, D_HID), lambda i: (i, 0))],
        out_shape=[jax.ShapeDtypeStruct((N_NODES, D_HID), jnp.float32)] * 2,
    )(aggp, cntp, Q1, W2l, W2r, b2d)


def _final(aggp, cntp, Q2, Wpad, bpad):
    def tc_body(a_ref, c_ref, q_ref, w_ref, b_ref, o_ref):
        a = a_ref[0] + a_ref[1]
        cnt = c_ref[0, :, 0:1] + c_ref[1, :, 0:1]
        inv = 1.0 / jnp.maximum(cnt, 1.0)
        h = jnp.maximum(a * inv + q_ref[...], 0.0)
        o_ref[...] = jnp.dot(h, w_ref[...],
                             preferred_element_type=jnp.float32,
                             precision=lax.Precision.HIGHEST) + b_ref[...]

    return pl.pallas_call(
        tc_body,
        grid=(N_NODES // ROW_BLK,),
        in_specs=[pl.BlockSpec((NC, ROW_BLK, D_HID), lambda i: (0, i, 0)),
                  pl.BlockSpec((NC, ROW_BLK, CW), lambda i: (0, i, 0)),
                  pl.BlockSpec((ROW_BLK, D_HID), lambda i: (i, 0)),
                  pl.BlockSpec((D_HID, 128), lambda i: (0, 0)),
                  pl.BlockSpec((1, 128), lambda i: (0, 0))],
        out_specs=pl.BlockSpec((ROW_BLK, 128), lambda i: (i, 0)),
        out_shape=jax.ShapeDtypeStruct((N_NODES, 128), jnp.float32),
    )(aggp, cntp, Q2, Wpad, bpad)


def kernel(x, edge_index, W1l, b1l, W1r, b1r, W2l, b2l, W2r, b2r, Wlin, blin):
    f32 = jnp.float32
    ei = edge_index.astype(jnp.int32)
    ones = jnp.ones((CHUNK, CW), f32)
    z64 = jnp.zeros((ROWS_PER_TILE, D_HID), f32)
    z16 = jnp.zeros((ROWS_PER_TILE, CW), f32)

    P1, Q1 = _dense2(x, W1l, W1r, (b1l + b1r).reshape(1, -1))
    agg1, cntp = _sc_scatter(True)(P1, ei, ones, z64, z16)
    P2, Q2 = _mid(agg1, cntp, Q1, W2l, W2r, (b2l + b2r).reshape(1, -1))
    agg2 = _sc_scatter(False)(P2, ei, z64)
    Wpad = jnp.pad(Wlin, ((0, 0), (0, 128 - D_OUT)))
    bpad = jnp.pad(blin, (0, 128 - D_OUT)).reshape(1, -1)
    outp = _final(agg2, cntp, Q2, Wpad, bpad)
    return outp[:, :D_OUT]


# speedup vs baseline: 1.0728x; 1.0728x over previous
import functools

import jax
import jax.numpy as jnp
from jax import lax
from jax.experimental import pallas as pl
from jax.experimental.pallas import tpu as pltpu
from jax.experimental.pallas import tpu_sc as plsc

N_NODES = 10000
N_EDGES = 320000
D_IN = 128
D_HID = 64
D_OUT = 2

NC = 2
NS = 16
NW = NC * NS
CHUNK = 128
EDGES_PER_TILE = N_EDGES // NW
FULL_CHUNKS = EDGES_PER_TILE // CHUNK
TAIL = EDGES_PER_TILE - FULL_CHUNKS * CHUNK
N_PAD = 10112
ROWS_PER_TILE = N_PAD // NS
CW = 16
ROW_BLK = 2000


def _sc_scatter(with_cnt):
    mesh = plsc.VectorSubcoreMesh(core_axis_name="c", subcore_axis_name="s")
    agg_t = jax.ShapeDtypeStruct((NC, N_PAD, D_HID), jnp.float32)
    out_type = [agg_t] if with_cnt else agg_t
    scratch = (
        [pltpu.VMEM((EDGES_PER_TILE,), jnp.int32),
         pltpu.VMEM((EDGES_PER_TILE,), jnp.int32)]
        + [pltpu.VMEM((CHUNK, D_HID), jnp.float32)] * 4
        + [pltpu.VMEM((TAIL, D_HID), jnp.float32),
           pltpu.VMEM_SHARED((N_PAD, D_HID), jnp.float32)]
        + [pltpu.SemaphoreType.DMA] * 8
    )
    if with_cnt:
        out_type.append(jax.ShapeDtypeStruct((NC, N_PAD, CW), jnp.float32))
        scratch += [
            pltpu.VMEM((CHUNK, CW), jnp.float32),
            pltpu.VMEM_SHARED((N_PAD, CW), jnp.float32),
            pltpu.SemaphoreType.DMA,
            pltpu.SemaphoreType.DMA,
        ]

    def body(*refs):
        if with_cnt:
            (p, ei, ones_h, z64, z16, agg_o, cnt_o,
             sv, dv, r0_, r1_, r2_, r3_, rT, acc,
             g0, g1, g2, g3, s0, s1, s2, s3,
             ones_v, cacc, semCA, semCB) = refs
        else:
            (p, ei, z64, agg_o,
             sv, dv, r0_, r1_, r2_, r3_, rT, acc,
             g0, g1, g2, g3, s0, s1, s2, s3) = refs
        rv = [r0_, r1_, r2_, r3_]
        gs = [g0, g1, g2, g3]
        ss = [s0, s1, s2, s3]
        cs = None if not with_cnt else [semCA, semCB]

        cid = lax.axis_index("c")
        sid = lax.axis_index("s")
        wid = cid * NS + sid
        r0 = sid * ROWS_PER_TILE
        base = wid * EDGES_PER_TILE

        pltpu.sync_copy(z64, acc.at[pl.ds(r0, ROWS_PER_TILE)])
        if with_cnt:
            pltpu.sync_copy(z16, cacc.at[pl.ds(r0, ROWS_PER_TILE)])
            pltpu.sync_copy(ones_h, ones_v)
        pltpu.sync_copy(ei.at[0].at[pl.ds(base, EDGES_PER_TILE)], sv)
        pltpu.sync_copy(ei.at[1].at[pl.ds(base, EDGES_PER_TILE)], dv)
        plsc.subcore_barrier()

        def g_start(j, b):
            pltpu.async_copy(p.at[sv.at[pl.ds(j * CHUNK, CHUNK)]],
                             rv[b], gs[b])

        def g_drain(j, b):
            pltpu.make_async_copy(p.at[sv.at[pl.ds(j * CHUNK, CHUNK)]],
                                  rv[b], gs[b]).wait()

        def r_fire(j, b):
            pltpu.async_copy(rv[b], acc.at[dv.at[pl.ds(j * CHUNK, CHUNK)]],
                             ss[b], add=True)

        def r_wait(j, b):
            pltpu.make_async_copy(rv[b],
                                  acc.at[dv.at[pl.ds(j * CHUNK, CHUNK)]],
                                  ss[b]).wait()

        def c_fire(j, m):
            pltpu.async_copy(ones_v, cacc.at[dv.at[pl.ds(j * CHUNK, CHUNK)]],
                             cs[m], add=True)

        def c_wait(j, m):
            pltpu.make_async_copy(
                ones_v, cacc.at[dv.at[pl.ds(j * CHUNK, CHUNK)]], cs[m]).wait()

        tail_ds = pl.ds(FULL_CHUNKS * CHUNK, TAIL)

        g_start(0, 0)
        g_start(1, 1)
        if with_cnt:
            c_fire(0, 0)
            c_fire(1, 1)
        g_drain(0, 0)
        r_fire(0, 0)
        g_start(2, 2)
        g_drain(1, 1)
        r_fire(1, 1)
        g_start(3, 3)

        def step(j, b, m):
            g_drain(j, b)
            r_fire(j, b)
            if with_cnt:
                c_wait(j - 2, m)
                c_fire(j, m)
            r_wait(j - 2, (b + 2) % 4)
            g_start(j + 2, (b + 2) % 4)

        @pl.loop(0, (FULL_CHUNKS - 6) // 4)
        def _(it):
            j0 = 2 + it * 4
            for k in range(4):
                step(j0 + k, (2 + k) % 4, k % 2)

        jE = FULL_CHUNKS - 4
        g_drain(jE, 2)
        r_fire(jE, 2)
        if with_cnt:
            c_wait(jE - 2, 0)
            c_fire(jE, 0)
        r_wait(jE - 2, 0)
        g_start(jE + 2, 0)

        g_drain(jE + 1, 3)
        r_fire(jE + 1, 3)
        if with_cnt:
            c_wait(jE - 1, 1)
            c_fire(jE + 1, 1)
        r_wait(jE - 1, 1)
        g_start(jE + 3, 1)

        g_drain(jE + 2, 0)
        r_fire(jE + 2, 0)
        if with_cnt:
            c_wait(jE, 0)
            c_fire(jE + 2, 0)
        r_wait(jE, 2)
        pltpu.async_copy(p.at[sv.at[tail_ds]], rT, gs[2])

        g_drain(jE + 3, 1)
        r_fire(jE + 3, 1)
        if with_cnt:
            c_wait(jE + 1, 1)
            c_fire(jE + 3, 1)
        r_wait(jE + 1, 3)

        pltpu.make_async_copy(p.at[sv.at[tail_ds]], rT, gs[2]).wait()
        pltpu.sync_copy(rT, acc.at[dv.at[tail_ds]], add=True)
        if with_cnt:
            c_wait(jE + 2, 0)
            c_wait(jE + 3, 1)
            pltpu.sync_copy(ones_v.at[pl.ds(0, TAIL)],
                            cacc.at[dv.at[tail_ds]], add=True)
        r_wait(jE + 2, 0)
        r_wait(jE + 3, 1)

        plsc.subcore_barrier()
        pltpu.sync_copy(acc.at[pl.ds(r0, ROWS_PER_TILE)],
                        agg_o.at[cid].at[pl.ds(r0, ROWS_PER_TILE)])
        if with_cnt:
            pltpu.sync_copy(cacc.at[pl.ds(r0, ROWS_PER_TILE)],
                            cnt_o.at[cid].at[pl.ds(r0, ROWS_PER_TILE)])

    cp = pltpu.CompilerParams(use_tc_tiling_on_sc=False)
    return pl.kernel(body, out_type=out_type, mesh=mesh, scratch_types=scratch,
                     compiler_params=cp)


def _dense2(x, Wl, Wr, b2d):
    def tc_body(x_ref, wl_ref, wr_ref, b_ref, p_ref, q_ref):
        xb = x_ref[...]
        p_ref[...] = jnp.dot(xb, wl_ref[...],
                             preferred_element_type=jnp.float32,
                             precision=lax.Precision.HIGHEST)
        q_ref[...] = jnp.dot(xb, wr_ref[...],
                             preferred_element_type=jnp.float32,
                             precision=lax.Precision.HIGHEST) + b_ref[...]

    return pl.pallas_call(
        tc_body,
        grid=(N_NODES // ROW_BLK,),
        in_specs=[pl.BlockSpec((ROW_BLK, D_IN), lambda i: (i, 0)),
                  pl.BlockSpec((D_IN, D_HID), lambda i: (0, 0)),
                  pl.BlockSpec((D_IN, D_HID), lambda i: (0, 0)),
                  pl.BlockSpec((1, D_HID), lambda i: (0, 0))],
        out_specs=[pl.BlockSpec((ROW_BLK, D_HID), lambda i: (i, 0)),
                   pl.BlockSpec((ROW_BLK, D_HID), lambda i: (i, 0))],
        out_shape=[jax.ShapeDtypeStruct((N_NODES, D_HID), jnp.float32)] * 2,
    )(x, Wl, Wr, b2d)


def _mid(aggp, cntp, Q1, W2l, W2r, b2d):
    def tc_body(a_ref, c_ref, q_ref, wl_ref, wr_ref, b_ref, p_ref, q2_ref):
        a = a_ref[0] + a_ref[1]
        cnt = c_ref[0, :, 0:1] + c_ref[1, :, 0:1]
        inv = 1.0 / jnp.maximum(cnt, 1.0)
        h = jnp.maximum(a * inv + q_ref[...], 0.0)
        p_ref[...] = jnp.dot(h, wl_ref[...],
                             preferred_element_type=jnp.float32,
                             precision=lax.Precision.HIGHEST)
        q2_ref[...] = jnp.dot(h, wr_ref[...],
                              preferred_element_type=jnp.float32,
                              precision=lax.Precision.HIGHEST) + b_ref[...]

    return pl.pallas_call(
        tc_body,
        grid=(N_NODES // ROW_BLK,),
        in_specs=[pl.BlockSpec((NC, ROW_BLK, D_HID), lambda i: (0, i, 0)),
                  pl.BlockSpec((NC, ROW_BLK, CW), lambda i: (0, i, 0)),
                  pl.BlockSpec((ROW_BLK, D_HID), lambda i: (i, 0)),
                  pl.BlockSpec((D_HID, D_HID), lambda i: (0, 0)),
                  pl.BlockSpec((D_HID, D_HID), lambda i: (0, 0)),
                  pl.BlockSpec((1, D_HID), lambda i: (0, 0))],
        out_specs=[pl.BlockSpec((ROW_BLK, D_HID), lambda i: (i, 0)),
                   pl.BlockSpec((ROW_BLK, D_HID), lambda i: (i, 0))],
        out_shape=[jax.ShapeDtypeStruct((N_NODES, D_HID), jnp.float32)] * 2,
    )(aggp, cntp, Q1, W2l, W2r, b2d)


def _final(aggp, cntp, Q2, Wpad, bpad):
    def tc_body(a_ref, c_ref, q_ref, w_ref, b_ref, o_ref):
        a = a_ref[0] + a_ref[1]
        cnt = c_ref[0, :, 0:1] + c_ref[1, :, 0:1]
        inv = 1.0 / jnp.maximum(cnt, 1.0)
        h = jnp.maximum(a * inv + q_ref[...], 0.0)
        o_ref[...] = jnp.dot(h, w_ref[...],
                             preferred_element_type=jnp.float32,
                             precision=lax.Precision.HIGHEST) + b_ref[...]

    return pl.pallas_call(
        tc_body,
        grid=(N_NODES // ROW_BLK,),
        in_specs=[pl.BlockSpec((NC, ROW_BLK, D_HID), lambda i: (0, i, 0)),
                  pl.BlockSpec((NC, ROW_BLK, CW), lambda i: (0, i, 0)),
                  pl.BlockSpec((ROW_BLK, D_HID), lambda i: (i, 0)),
                  pl.BlockSpec((D_HID, 128), lambda i: (0, 0)),
                  pl.BlockSpec((1, 128), lambda i: (0, 0))],
        out_specs=pl.BlockSpec((ROW_BLK, 128), lambda i: (i, 0)),
        out_shape=jax.ShapeDtypeStruct((N_NODES, 128), jnp.float32),
    )(aggp, cntp, Q2, Wpad, bpad)


def kernel(x, edge_index, W1l, b1l, W1r, b1r, W2l, b2l, W2r, b2r, Wlin, blin):
    f32 = jnp.float32
    ei = edge_index.astype(jnp.int32)
    ones = jnp.ones((CHUNK, CW), f32)
    z64 = jnp.zeros((ROWS_PER_TILE, D_HID), f32)
    z16 = jnp.zeros((ROWS_PER_TILE, CW), f32)

    P1, Q1 = _dense2(x, W1l, W1r, (b1l + b1r).reshape(1, -1))
    agg1, cntp = _sc_scatter(True)(P1, ei, ones, z64, z16)
    P2, Q2 = _mid(agg1, cntp, Q1, W2l, W2r, (b2l + b2r).reshape(1, -1))
    agg2 = _sc_scatter(False)(P2, ei, z64)
    Wpad = jnp.pad(Wlin, ((0, 0), (0, 128 - D_OUT)))
    bpad = jnp.pad(blin, (0, 128 - D_OUT)).reshape(1, -1)
    outp = _final(agg2, cntp, Q2, Wpad, bpad)
    return outp[:, :D_OUT]
